# baseline (speedup 1.0000x reference)
import functools

import jax
import jax.numpy as jnp
from jax import lax
from jax.experimental import pallas as pl
from jax.experimental.pallas import tpu as pltpu
from jax.experimental.pallas import tpu_sc as plsc

_B = 16384
_D = 16
_NW = 32
_BPW = _B // _NW
_CHUNK = 128
_NCH = _BPW // _CHUNK
_WD = 1e-05


def _vsqrt(x):
    i = lax.bitcast_convert_type(x, jnp.int32)
    y = lax.bitcast_convert_type(jnp.int32(0x5F3759DF) - (i >> 1), jnp.float32)
    for _ in range(3):
        y = y * (1.5 - 0.5 * x * y * y)
    return x * y


_mesh = plsc.VectorSubcoreMesh(core_axis_name="c", subcore_axis_name="s")


@functools.partial(
    pl.kernel,
    out_type=[
        jax.ShapeDtypeStruct((_B, _D), jnp.float32),
        jax.ShapeDtypeStruct((_B,), jnp.float32),
    ],
    mesh=_mesh,
    compiler_params=pltpu.CompilerParams(needs_layout_passes=False,
                                         use_tc_tiling_on_sc=False),
    scratch_types=[
        pltpu.VMEM((_NCH, _CHUNK), jnp.int32),
        pltpu.VMEM((_NCH, _CHUNK), jnp.int32),
        pltpu.VMEM((_BPW, _D), jnp.float32),
        pltpu.VMEM((_BPW, _D), jnp.float32),
        pltpu.VMEM((_BPW, _D), jnp.float32),
        pltpu.VMEM((_BPW,), jnp.float32),
        pltpu.SemaphoreType.DMA,
    ],
)
def _sc_embed(idxx_hbm, idxa_hbm, xt_hbm, at_hbm, ox_hbm, owd_hbm,
              idxx_v, idxa_v, xrows, arows, prod, wd_v, sem):
    cid = lax.axis_index("c")
    sid = lax.axis_index("s")
    wid = sid * 2 + cid
    base = wid * _BPW

    for ch in range(_NCH):
        pltpu.sync_copy(idxx_hbm.at[pl.ds(base + ch * _CHUNK, _CHUNK)],
                        idxx_v.at[ch])
        pltpu.sync_copy(idxa_hbm.at[pl.ds(base + ch * _CHUNK, _CHUNK)],
                        idxa_v.at[ch])
    copies = []
    for ch in range(_NCH):
        copies.append(pltpu.async_copy(
            xt_hbm.at[idxx_v.at[ch]],
            xrows.at[pl.ds(ch * _CHUNK, _CHUNK)], sem))
        copies.append(pltpu.async_copy(
            at_hbm.at[idxa_v.at[ch]],
            arows.at[pl.ds(ch * _CHUNK, _CHUNK)], sem))
    for cp in copies:
        cp.wait()

    iota = lax.iota(jnp.int32, 16)

    def block(r, carry):
        rb = r * 16
        for k in range(16):
            i = rb + k
            prod[i, :] = xrows[i, :] * arows[i, :]
        row_idx = rb + iota
        accx = jnp.zeros((16,), jnp.float32)
        acca = jnp.zeros((16,), jnp.float32)
        for j in range(_D):
            cj = jnp.full((16,), j, jnp.int32)
            gx = plsc.load_gather(xrows, [row_idx, cj])
            ga = plsc.load_gather(arows, [row_idx, cj])
            accx = accx + gx * gx
            acca = acca + ga * ga
        wd_v[pl.ds(rb, 16)] = _WD * (_vsqrt(accx) + _vsqrt(acca))
        return carry

    lax.fori_loop(0, _BPW // 16, block, 0)

    pltpu.sync_copy(prod, ox_hbm.at[pl.ds(base, _BPW)])
    pltpu.sync_copy(wd_v, owd_hbm.at[pl.ds(base, _BPW)])


def kernel(x_raw, x_table, a_table):
    idx = x_raw.astype(jnp.int32)
    out_x, out_wd = _sc_embed(idx[:, 0], idx[:, 1], x_table, a_table)
    return (out_x, out_wd)
